# Initial kernel scaffold; baseline (speedup 1.0000x reference)
import jax
import jax.numpy as jnp
from jax.experimental import pallas as pl


def kernel(input, adj, W, b):
    raise NotImplementedError("write your pallas kernel here")


# speedup vs baseline: 1.3375x; 1.3375x over previous
import jax
import jax.numpy as jnp
from jax.experimental import pallas as pl
from jax.experimental.pallas import tpu as pltpu

_TILE_N = 512


def _gcn_body(x_ref, adj_ref, w_ref, bias_ref, out_ref, support_ref):
    i = pl.program_id(1)

    @pl.when(i == 0)
    def _():
        support_ref[...] = jnp.dot(
            x_ref[0], w_ref[...], preferred_element_type=jnp.float32
        )

    out_ref[0] = (
        jnp.dot(adj_ref[0], support_ref[...], preferred_element_type=jnp.float32)
        + bias_ref[...]
    )


def kernel(input, adj, W, b):
    batch, n, d_in = input.shape
    d_out = W.shape[1]
    tile_n = min(_TILE_N, n)
    grid = (batch, n // tile_n)
    return pl.pallas_call(
        _gcn_body,
        grid=grid,
        in_specs=[
            pl.BlockSpec((1, n, d_in), lambda bb, i: (bb, 0, 0)),
            pl.BlockSpec((1, tile_n, n), lambda bb, i: (bb, i, 0)),
            pl.BlockSpec((d_in, d_out), lambda bb, i: (0, 0)),
            pl.BlockSpec((1, d_out), lambda bb, i: (0, 0)),
        ],
        out_specs=pl.BlockSpec((1, tile_n, d_out), lambda bb, i: (bb, i, 0)),
        out_shape=jax.ShapeDtypeStruct((batch, n, d_out), jnp.float32),
        scratch_shapes=[pltpu.VMEM((n, d_out), jnp.float32)],
    )(input, adj, W, b.reshape(1, d_out))


# speedup vs baseline: 1.3391x; 1.0012x over previous
import jax
import jax.numpy as jnp
from jax.experimental import pallas as pl
from jax.experimental.pallas import tpu as pltpu

_TILE_N = 512


def _gcn_body(x_ref, adj_ref, w_ref, bias_ref, out_ref, support_ref):
    i = pl.program_id(1)

    @pl.when(i == 0)
    def _():
        support_ref[...] = jnp.dot(
            x_ref[0], w_ref[...], preferred_element_type=jnp.float32
        ).astype(jnp.bfloat16)

    out_ref[0] = (
        jnp.dot(
            adj_ref[0].astype(jnp.bfloat16),
            support_ref[...],
            preferred_element_type=jnp.float32,
        )
        + bias_ref[...]
    )


def kernel(input, adj, W, b):
    batch, n, d_in = input.shape
    d_out = W.shape[1]
    tile_n = min(_TILE_N, n)
    grid = (batch, n // tile_n)
    return pl.pallas_call(
        _gcn_body,
        grid=grid,
        in_specs=[
            pl.BlockSpec((1, n, d_in), lambda bb, i: (bb, 0, 0)),
            pl.BlockSpec((1, tile_n, n), lambda bb, i: (bb, i, 0)),
            pl.BlockSpec((d_in, d_out), lambda bb, i: (0, 0)),
            pl.BlockSpec((1, d_out), lambda bb, i: (0, 0)),
        ],
        out_specs=pl.BlockSpec((1, tile_n, d_out), lambda bb, i: (bb, i, 0)),
        out_shape=jax.ShapeDtypeStruct((batch, n, d_out), jnp.float32),
        scratch_shapes=[pltpu.VMEM((n, d_out), jnp.bfloat16)],
    )(input, adj, W, b.reshape(1, d_out))
